# baseline (speedup 1.0000x reference)
import functools

import jax
import jax.numpy as jnp
from jax import lax
from jax.experimental import pallas as pl
from jax.experimental.pallas import tpu as pltpu
from jax.experimental.pallas import tpu_sc as plsc

_SC_CORES = 2
_SC_SUBCORES = 16
_SC_WORKERS = _SC_CORES * _SC_SUBCORES


def _router_body(x_ref, gw_ref, pos_ref, wtb_ref, offs_ref):
    T = x_ref.shape[0]
    E = gw_ref.shape[0]
    x = x_ref[...]
    gw = gw_ref[...]
    logits = jax.lax.dot_general(x, gw, (((1,), (1,)), ((), ())),
                                 preferred_element_type=jnp.float32)
    m = jnp.max(logits, axis=1, keepdims=True)
    s = jnp.sum(jnp.exp(logits - m), axis=1, keepdims=True)
    topv = 1.0 / s
    wt = topv / (topv + 1e-8)

    lane = jax.lax.broadcasted_iota(jnp.int32, (T, E), 1)
    cand = jnp.where(logits >= m, lane, E)
    topi = jnp.min(cand, axis=1, keepdims=True)

    onehot = (lane == topi).astype(jnp.bfloat16)
    ri = jax.lax.broadcasted_iota(jnp.int32, (T, T), 0)
    ci = jax.lax.broadcasted_iota(jnp.int32, (T, T), 1)
    tril = (ci <= ri).astype(jnp.bfloat16)
    run = jax.lax.dot_general(tril, onehot, (((1,), (0,)), ((), ())),
                              preferred_element_type=jnp.float32)
    counts = run[T - 1:T, :]
    ei = jax.lax.broadcasted_iota(jnp.int32, (E, E), 0)
    ej = jax.lax.broadcasted_iota(jnp.int32, (E, E), 1)
    sut = (ei < ej).astype(jnp.float32)
    excl = jax.lax.dot_general(counts, sut, (((1,), (0,)), ((), ())),
                               preferred_element_type=jnp.float32,
                               precision=jax.lax.Precision.HIGHEST)
    onef = onehot.astype(jnp.float32)
    posf = jnp.sum(onef * (excl + run - 1.0), axis=1, keepdims=True)
    pos_ref[...] = posf.astype(jnp.int32)
    wtb_ref[...] = jnp.broadcast_to(wt, (T, 128))
    offs_ref[...] = jnp.concatenate(
        [excl.astype(jnp.int32), jnp.full((1, 128 - E), T, jnp.int32)], axis=1)


def _permute_body(pos_ref, src_ref, srcw_ref, dst_ref, dstw_ref, *, invert):
    T = src_ref.shape[0]
    posv = pos_ref[...]
    lane = jax.lax.broadcasted_iota(jnp.int32, (T, T), 1)
    oh = (lane == posv).astype(jnp.bfloat16)
    if invert:
        dims = (((1,), (0,)), ((), ()))
    else:
        dims = (((0,), (0,)), ((), ()))

    def apply(v):
        hi = v.astype(jnp.bfloat16)
        lo = (v - hi.astype(jnp.float32)).astype(jnp.bfloat16)
        out = jax.lax.dot_general(oh, hi, dims, preferred_element_type=jnp.float32)
        out += jax.lax.dot_general(oh, lo, dims, preferred_element_type=jnp.float32)
        return out

    dst_ref[...] = apply(src_ref[...])
    if srcw_ref is not None:
        dstw_ref[...] = apply(srcw_ref[...])


def _sc_worker_base(chunk):
    wid = lax.axis_index("s") * _SC_CORES + lax.axis_index("c")
    return wid * chunk


def _dispatch_sc_body(pos_hbm, x_hbm, wtb_hbm, sx_hbm, swtb_hbm,
                      idx_v, rows_v, wrows_v, sem, *, chunk):
    base = _sc_worker_base(chunk)
    pltpu.sync_copy(pos_hbm.at[pl.ds(base, chunk)], idx_v)
    pltpu.sync_copy(x_hbm.at[pl.ds(base, chunk)], rows_v)
    pltpu.sync_copy(wtb_hbm.at[pl.ds(base, chunk)], wrows_v)
    pltpu.async_copy(rows_v, sx_hbm.at[idx_v], sem).wait()
    pltpu.async_copy(wrows_v, swtb_hbm.at[idx_v], sem).wait()


def _combine_sc_body(pos_hbm, comb_hbm, out_hbm, idx_v, rows_v, sem, *,
                     chunk):
    base = _sc_worker_base(chunk)
    pltpu.sync_copy(pos_hbm.at[pl.ds(base, chunk)], idx_v)
    pltpu.async_copy(comb_hbm.at[idx_v], rows_v, sem).wait()
    pltpu.sync_copy(rows_v, out_hbm.at[pl.ds(base, chunk)])


def _gelu(v):
    return v * 0.5 * (1.0 + jax.lax.erf(v * 0.7071067811865476))


def _ffn_body(offs_ref, xs_ref, swt_ref, w1_ref, b1_ref, w2_ref, b2_ref,
              out_ref, *, tm, eb):
    T = xs_ref.shape[0]
    g = pl.program_id(0)
    for j in range(eb):
        _ffn_expert(g * eb + j, offs_ref, xs_ref, swt_ref, w1_ref[j],
                    b1_ref[j], w2_ref[j], b2_ref[j], out_ref, tm=tm)


def _ffn_expert(e, offs_ref, xs_ref, swt_ref, w1, b1, w2, b2, out_ref, *, tm):
    T = xs_ref.shape[0]
    start = offs_ref[e]
    end = offs_ref[e + 1]
    astart = (start // 8) * 8
    nblk = (end - astart + tm - 1) // tm
    w1h = w1.astype(jnp.bfloat16)
    w2h = w2.astype(jnp.bfloat16)
    dn = (((1,), (1,)), ((), ()))

    def body(i, carry):
        sidx = pl.multiple_of(jnp.minimum(astart + i * tm, T - tm), 8)
        xb = xs_ref[pl.ds(sidx, tm), :].astype(jnp.bfloat16)
        h = jax.lax.dot_general(xb, w1h, dn,
                                preferred_element_type=jnp.float32) + b1
        h = _gelu(h)
        hb = h.astype(jnp.bfloat16)
        eo = jax.lax.dot_general(hb, w2h, dn,
                                 preferred_element_type=jnp.float32) + b2
        eo = eo * swt_ref[pl.ds(sidx, tm), 0:1]
        rid = sidx + jax.lax.broadcasted_iota(jnp.int32, (tm, 1), 0)
        mask = (rid >= start) & (rid < end)
        cur = out_ref[pl.ds(sidx, tm), :]
        out_ref[pl.ds(sidx, tm), :] = jnp.where(mask, eo, cur)
        return carry

    jax.lax.fori_loop(0, nblk, body, 0)


_TM = 32
_EB = 2


@jax.jit
def kernel(x, gate_w, w1, b1, w2, b2):
    B, N, C = x.shape
    T = B * N
    E, HID = b1.shape
    flat = x.reshape(T, C)

    pos, wtb, offs128 = pl.pallas_call(
        _router_body,
        out_shape=[
            jax.ShapeDtypeStruct((T, 1), jnp.int32),
            jax.ShapeDtypeStruct((T, 128), jnp.float32),
            jax.ShapeDtypeStruct((1, 128), jnp.int32),
        ],
    )(flat, gate_w)
    offs = offs128.reshape(-1)[:E + 1]
    pos1d = pos.reshape(T)

    chunk = T // _SC_WORKERS
    mesh = plsc.VectorSubcoreMesh(core_axis_name="c", subcore_axis_name="s")
    sorted_x, sorted_wtb = pl.kernel(
        functools.partial(_dispatch_sc_body, chunk=chunk),
        out_type=[
            jax.ShapeDtypeStruct((T, C), jnp.float32),
            jax.ShapeDtypeStruct((T, 128), jnp.float32),
        ],
        mesh=mesh,
        scratch_types=[
            pltpu.VMEM((chunk,), jnp.int32),
            pltpu.VMEM((chunk, C), jnp.float32),
            pltpu.VMEM((chunk, 128), jnp.float32),
            pltpu.SemaphoreType.DMA,
        ],
    )(pos1d, flat, wtb)

    combined = pl.pallas_call(
        functools.partial(_ffn_body, tm=_TM, eb=_EB),
        grid_spec=pltpu.PrefetchScalarGridSpec(
            num_scalar_prefetch=1,
            grid=(E // _EB,),
            in_specs=[
                pl.BlockSpec((T, C), lambda e, offs: (0, 0)),
                pl.BlockSpec((T, 128), lambda e, offs: (0, 0)),
                pl.BlockSpec((_EB, HID, C), lambda e, offs: (e, 0, 0)),
                pl.BlockSpec((_EB, 1, HID), lambda e, offs: (e, 0, 0)),
                pl.BlockSpec((_EB, C, HID), lambda e, offs: (e, 0, 0)),
                pl.BlockSpec((_EB, 1, C), lambda e, offs: (e, 0, 0)),
            ],
            out_specs=pl.BlockSpec((T, C), lambda e, offs: (0, 0)),
        ),
        out_shape=jax.ShapeDtypeStruct((T, C), jnp.float32),
    )(offs, sorted_x, sorted_wtb, w1, b1.reshape(E, 1, HID), w2,
      b2.reshape(E, 1, C))

    out = pl.kernel(
        functools.partial(_combine_sc_body, chunk=chunk),
        out_type=jax.ShapeDtypeStruct((T, C), jnp.float32),
        mesh=mesh,
        scratch_types=[
            pltpu.VMEM((chunk,), jnp.int32),
            pltpu.VMEM((chunk, C), jnp.float32),
            pltpu.SemaphoreType.DMA,
        ],
    )(pos1d, combined)
    return out.reshape(B, N, C)


# speedup vs baseline: 1.1417x; 1.1417x over previous
import functools

import jax
import jax.numpy as jnp
from jax import lax
from jax.experimental import pallas as pl
from jax.experimental.pallas import tpu as pltpu
from jax.experimental.pallas import tpu_sc as plsc

_SC_CORES = 2
_SC_SUBCORES = 16
_SC_WORKERS = _SC_CORES * _SC_SUBCORES


def _router_body(x_ref, gw_ref, pos_ref, wtb_ref, offs_ref):
    T = x_ref.shape[0]
    E = gw_ref.shape[0]
    x = x_ref[...]
    gw = gw_ref[...]
    logits = jax.lax.dot_general(x, gw, (((1,), (1,)), ((), ())),
                                 preferred_element_type=jnp.float32)
    m = jnp.max(logits, axis=1, keepdims=True)
    s = jnp.sum(jnp.exp(logits - m), axis=1, keepdims=True)
    topv = 1.0 / s
    wt = topv / (topv + 1e-8)

    lane = jax.lax.broadcasted_iota(jnp.int32, (T, E), 1)
    cand = jnp.where(logits >= m, lane, E)
    topi = jnp.min(cand, axis=1, keepdims=True)

    onehot = (lane == topi).astype(jnp.bfloat16)
    CH = 128
    NCH = T // CH
    ri = jax.lax.broadcasted_iota(jnp.int32, (CH, CH), 0)
    ci = jax.lax.broadcasted_iota(jnp.int32, (CH, CH), 1)
    tril = (ci <= ri).astype(jnp.bfloat16)
    dn_lo = (((1,), (0,)), ((), ()))
    runs = [jax.lax.dot_general(tril, onehot[k * CH:(k + 1) * CH, :], dn_lo,
                                preferred_element_type=jnp.float32)
            for k in range(NCH)]
    totals = jnp.concatenate([r[CH - 1:CH, :] for r in runs], axis=0)
    ki = jax.lax.broadcasted_iota(jnp.int32, (NCH, NCH), 0)
    kj = jax.lax.broadcasted_iota(jnp.int32, (NCH, NCH), 1)
    strl = (ki < kj).astype(jnp.float32)
    cpre = jax.lax.dot_general(strl, totals, (((0,), (0,)), ((), ())),
                               preferred_element_type=jnp.float32,
                               precision=jax.lax.Precision.HIGHEST)
    counts = cpre[NCH - 1:NCH, :] + totals[NCH - 1:NCH, :]
    ei = jax.lax.broadcasted_iota(jnp.int32, (E, E), 0)
    ej = jax.lax.broadcasted_iota(jnp.int32, (E, E), 1)
    sut = (ei < ej).astype(jnp.float32)
    excl = jax.lax.dot_general(counts, sut, (((1,), (0,)), ((), ())),
                               preferred_element_type=jnp.float32,
                               precision=jax.lax.Precision.HIGHEST)
    onef = onehot.astype(jnp.float32)
    run = jnp.concatenate(
        [runs[k] + cpre[k:k + 1, :] for k in range(NCH)], axis=0)
    posf = jnp.sum(onef * (excl + run - 1.0), axis=1, keepdims=True)
    pos_ref[...] = posf.astype(jnp.int32)
    wtb_ref[...] = jnp.broadcast_to(wt, (T, 128))
    offs_ref[...] = jnp.concatenate(
        [excl.astype(jnp.int32), jnp.full((1, 128 - E), T, jnp.int32)], axis=1)


def _permute_body(pos_ref, src_ref, srcw_ref, dst_ref, dstw_ref, *, invert):
    T = src_ref.shape[0]
    posv = pos_ref[...]
    lane = jax.lax.broadcasted_iota(jnp.int32, (T, T), 1)
    oh = (lane == posv).astype(jnp.bfloat16)
    if invert:
        dims = (((1,), (0,)), ((), ()))
    else:
        dims = (((0,), (0,)), ((), ()))

    def apply(v):
        hi = v.astype(jnp.bfloat16)
        lo = (v - hi.astype(jnp.float32)).astype(jnp.bfloat16)
        out = jax.lax.dot_general(oh, hi, dims, preferred_element_type=jnp.float32)
        out += jax.lax.dot_general(oh, lo, dims, preferred_element_type=jnp.float32)
        return out

    dst_ref[...] = apply(src_ref[...])
    if srcw_ref is not None:
        dstw_ref[...] = apply(srcw_ref[...])


def _sc_worker_base(chunk):
    wid = lax.axis_index("s") * _SC_CORES + lax.axis_index("c")
    return wid * chunk


def _dispatch_sc_body(pos_hbm, x_hbm, wtb_hbm, sx_hbm, swtb_hbm,
                      idx_v, rows_v, wrows_v, sem, *, chunk):
    base = _sc_worker_base(chunk)
    pltpu.sync_copy(pos_hbm.at[pl.ds(base, chunk)], idx_v)
    pltpu.sync_copy(x_hbm.at[pl.ds(base, chunk)], rows_v)
    pltpu.sync_copy(wtb_hbm.at[pl.ds(base, chunk)], wrows_v)
    pltpu.async_copy(rows_v, sx_hbm.at[idx_v], sem).wait()
    pltpu.async_copy(wrows_v, swtb_hbm.at[idx_v], sem).wait()


def _combine_sc_body(pos_hbm, comb_hbm, out_hbm, idx_v, rows_v, sem, *,
                     chunk):
    base = _sc_worker_base(chunk)
    pltpu.sync_copy(pos_hbm.at[pl.ds(base, chunk)], idx_v)
    pltpu.async_copy(comb_hbm.at[idx_v], rows_v, sem).wait()
    pltpu.sync_copy(rows_v, out_hbm.at[pl.ds(base, chunk)])


def _gelu(v):
    return v * 0.5 * (1.0 + jax.lax.erf(v * 0.7071067811865476))


def _ffn_body(offs_ref, xs_ref, swt_ref, w1_ref, b1_ref, w2_ref, b2_ref,
              out_ref, *, tm, eb):
    T = xs_ref.shape[0]
    g = pl.program_id(0)
    for j in range(eb):
        _ffn_expert(g * eb + j, offs_ref, xs_ref, swt_ref, w1_ref[j],
                    b1_ref[j], w2_ref[j], b2_ref[j], out_ref, tm=tm)


def _ffn_expert(e, offs_ref, xs_ref, swt_ref, w1, b1, w2, b2, out_ref, *, tm):
    T = xs_ref.shape[0]
    start = offs_ref[e]
    end = offs_ref[e + 1]
    astart = (start // 8) * 8
    nblk = (end - astart + tm - 1) // tm
    w1h = w1.astype(jnp.bfloat16)
    w2h = w2.astype(jnp.bfloat16)
    dn = (((1,), (1,)), ((), ()))

    def body(i, carry):
        sidx = pl.multiple_of(jnp.minimum(astart + i * tm, T - tm), 8)
        xb = xs_ref[pl.ds(sidx, tm), :].astype(jnp.bfloat16)
        h = jax.lax.dot_general(xb, w1h, dn,
                                preferred_element_type=jnp.float32) + b1
        h = _gelu(h)
        hb = h.astype(jnp.bfloat16)
        eo = jax.lax.dot_general(hb, w2h, dn,
                                 preferred_element_type=jnp.float32) + b2
        eo = eo * swt_ref[pl.ds(sidx, tm), 0:1]
        rid = sidx + jax.lax.broadcasted_iota(jnp.int32, (tm, 1), 0)
        mask = (rid >= start) & (rid < end)
        cur = out_ref[pl.ds(sidx, tm), :]
        out_ref[pl.ds(sidx, tm), :] = jnp.where(mask, eo, cur)
        return carry

    jax.lax.fori_loop(0, nblk, body, 0)


_TM = 64
_EB = 2


@jax.jit
def kernel(x, gate_w, w1, b1, w2, b2):
    B, N, C = x.shape
    T = B * N
    E, HID = b1.shape
    flat = x.reshape(T, C)

    pos, wtb, offs128 = pl.pallas_call(
        _router_body,
        out_shape=[
            jax.ShapeDtypeStruct((T, 1), jnp.int32),
            jax.ShapeDtypeStruct((T, 128), jnp.float32),
            jax.ShapeDtypeStruct((1, 128), jnp.int32),
        ],
    )(flat, gate_w)
    offs = offs128.reshape(-1)
    pos1d = pos.reshape(T)

    chunk = T // _SC_WORKERS
    mesh = plsc.VectorSubcoreMesh(core_axis_name="c", subcore_axis_name="s")
    sorted_x, sorted_wtb = pl.kernel(
        functools.partial(_dispatch_sc_body, chunk=chunk),
        out_type=[
            jax.ShapeDtypeStruct((T, C), jnp.float32),
            jax.ShapeDtypeStruct((T, 128), jnp.float32),
        ],
        mesh=mesh,
        scratch_types=[
            pltpu.VMEM((chunk,), jnp.int32),
            pltpu.VMEM((chunk, C), jnp.float32),
            pltpu.VMEM((chunk, 128), jnp.float32),
            pltpu.SemaphoreType.DMA,
        ],
    )(pos1d, flat, wtb)

    combined = pl.pallas_call(
        functools.partial(_ffn_body, tm=_TM, eb=_EB),
        grid_spec=pltpu.PrefetchScalarGridSpec(
            num_scalar_prefetch=1,
            grid=(E // _EB,),
            in_specs=[
                pl.BlockSpec((T, C), lambda e, offs: (0, 0)),
                pl.BlockSpec((T, 128), lambda e, offs: (0, 0)),
                pl.BlockSpec((_EB, HID, C), lambda e, offs: (e, 0, 0)),
                pl.BlockSpec((_EB, 1, HID), lambda e, offs: (e, 0, 0)),
                pl.BlockSpec((_EB, C, HID), lambda e, offs: (e, 0, 0)),
                pl.BlockSpec((_EB, 1, C), lambda e, offs: (e, 0, 0)),
            ],
            out_specs=pl.BlockSpec((T, C), lambda e, offs: (0, 0)),
        ),
        out_shape=jax.ShapeDtypeStruct((T, C), jnp.float32),
    )(offs, sorted_x, sorted_wtb, w1, b1.reshape(E, 1, HID), w2,
      b2.reshape(E, 1, C))

    out = pl.kernel(
        functools.partial(_combine_sc_body, chunk=chunk),
        out_type=jax.ShapeDtypeStruct((T, C), jnp.float32),
        mesh=mesh,
        scratch_types=[
            pltpu.VMEM((chunk,), jnp.int32),
            pltpu.VMEM((chunk, C), jnp.float32),
            pltpu.SemaphoreType.DMA,
        ],
    )(pos1d, combined)
    return out.reshape(B, N, C)
